# baseline (speedup 1.0000x reference)
import functools

import jax
import jax.numpy as jnp
from jax import lax
from jax.experimental import pallas as pl
from jax.experimental.pallas import tpu as pltpu
from jax.experimental.pallas import tpu_sc as plsc

_B = 4096
_D = 128
_H1 = 1024
_H2 = 512
_H3 = 128


def _sc_gather(uid, gid, aid, oid, zid, utab, gtab, atab, otab, ztab):
    info = plsc.get_sparse_core_info()
    nc, ns = info.num_cores, info.num_subcores
    nw = nc * ns
    bpw = _B // nw

    mesh = plsc.VectorSubcoreMesh(core_axis_name="c", subcore_axis_name="s")
    out_t = [jax.ShapeDtypeStruct((_B, _D), jnp.float32) for _ in range(5)]
    scratch = (
        [pltpu.VMEM((bpw,), jnp.int32) for _ in range(5)]
        + [pltpu.VMEM((bpw, _D), jnp.float32) for _ in range(5)]
        + [pltpu.SemaphoreType.DMA]
    )

    @functools.partial(pl.kernel, mesh=mesh, out_type=out_t,
                       scratch_types=scratch)
    def gather(uid_h, gid_h, aid_h, oid_h, zid_h,
               utab_h, gtab_h, atab_h, otab_h, ztab_h,
               uo_h, go_h, ao_h, oo_h, zo_h,
               ui_v, gi_v, ai_v, oi_v, zi_v,
               ur_v, gr_v, ar_v, or_v, zr_v, sem):
        wid = lax.axis_index("s") * nc + lax.axis_index("c")
        base = wid * bpw
        sl = pl.ds(base, bpw)
        pltpu.sync_copy(uid_h.at[sl], ui_v)
        pltpu.sync_copy(gid_h.at[sl], gi_v)
        pltpu.sync_copy(aid_h.at[sl], ai_v)
        pltpu.sync_copy(oid_h.at[sl], oi_v)
        pltpu.sync_copy(zid_h.at[sl], zi_v)
        cu = pltpu.async_copy(utab_h.at[ui_v], ur_v, sem)
        cg = pltpu.async_copy(gtab_h.at[gi_v], gr_v, sem)
        ca = pltpu.async_copy(atab_h.at[ai_v], ar_v, sem)
        co = pltpu.async_copy(otab_h.at[oi_v], or_v, sem)
        cz = pltpu.async_copy(ztab_h.at[zi_v], zr_v, sem)
        cu.wait()
        cg.wait()
        ca.wait()
        co.wait()
        cz.wait()
        pltpu.sync_copy(ur_v, uo_h.at[sl])
        pltpu.sync_copy(gr_v, go_h.at[sl])
        pltpu.sync_copy(ar_v, ao_h.at[sl])
        pltpu.sync_copy(or_v, oo_h.at[sl])
        pltpu.sync_copy(zr_v, zo_h.at[sl])

    return gather(uid, gid, aid, oid, zid, utab, gtab, atab, otab, ztab)


def _mlp_body(u, g, a, o, z, w1, b1, w2, b2, w3, b3, out):
    x = jnp.concatenate([u[...], g[...], a[...], o[...], z[...]], axis=1)
    h = jnp.dot(x, w1[...], preferred_element_type=jnp.float32) + b1[...]
    h = jnp.maximum(h, 0.0)
    h = jnp.dot(h, w2[...], preferred_element_type=jnp.float32) + b2[...]
    h = jnp.maximum(h, 0.0)
    h = jnp.dot(h, w3[...], preferred_element_type=jnp.float32) + b3[...]
    h = jnp.maximum(h, 0.0)
    ss = jnp.sum(h * h, axis=1, keepdims=True)
    out[...] = h * lax.rsqrt(jnp.maximum(ss, 1e-24))


def _mlp(u, g, a, o, z, w1, b1, w2, b2, w3, b3):
    bb = 512
    const = lambda i: (0, 0)
    return pl.pallas_call(
        _mlp_body,
        grid=(_B // bb,),
        in_specs=[
            pl.BlockSpec((bb, _D), lambda i: (i, 0)),
            pl.BlockSpec((bb, _D), lambda i: (i, 0)),
            pl.BlockSpec((bb, _D), lambda i: (i, 0)),
            pl.BlockSpec((bb, _D), lambda i: (i, 0)),
            pl.BlockSpec((bb, _D), lambda i: (i, 0)),
            pl.BlockSpec((5 * _D, _H1), const),
            pl.BlockSpec((1, _H1), const),
            pl.BlockSpec((_H1, _H2), const),
            pl.BlockSpec((1, _H2), const),
            pl.BlockSpec((_H2, _H3), const),
            pl.BlockSpec((1, _H3), const),
        ],
        out_specs=pl.BlockSpec((bb, _D), lambda i: (i, 0)),
        out_shape=jax.ShapeDtypeStruct((_B, _D), jnp.float32),
    )(u, g, a, o, z, w1, b1.reshape(1, _H1), w2, b2.reshape(1, _H2),
      w3, b3.reshape(1, _H3))


def kernel(user_id, gender, age, occupation, zip_id, user_tab, gender_tab,
           age_tab, occ_tab, zip_tab, W1, b1, W2, b2, W3, b3):
    uid = user_id.astype(jnp.int32)
    gid = gender.astype(jnp.int32)
    aid = age.astype(jnp.int32)
    oid = occupation.astype(jnp.int32)
    zid = zip_id.astype(jnp.int32)
    u, g, a, o, z = _sc_gather(uid, gid, aid, oid, zid,
                               user_tab, gender_tab, age_tab, occ_tab,
                               zip_tab)
    return _mlp(u, g, a, o, z, W1, b1, W2, b2, W3, b3)


# speedup vs baseline: 3.3967x; 3.3967x over previous
import functools

import jax
import jax.numpy as jnp
from jax import lax
from jax.experimental import pallas as pl
from jax.experimental.pallas import tpu as pltpu
from jax.experimental.pallas import tpu_sc as plsc

_B = 4096
_D = 128
_H1 = 1024
_H2 = 512
_H3 = 128
_BB = 512


def _sc_gather(uid, zid, utab, ztab):
    info = plsc.get_sparse_core_info()
    nc, ns = info.num_cores, info.num_subcores
    nw = nc * ns
    bpw = _B // nw

    mesh = plsc.VectorSubcoreMesh(core_axis_name="c", subcore_axis_name="s")
    out_t = [jax.ShapeDtypeStruct((_B, _D), jnp.float32) for _ in range(2)]
    scratch = (
        [pltpu.VMEM((bpw,), jnp.int32) for _ in range(2)]
        + [pltpu.VMEM((bpw, _D), jnp.float32) for _ in range(2)]
        + [pltpu.SemaphoreType.DMA, pltpu.SemaphoreType.DMA]
    )

    @functools.partial(pl.kernel, mesh=mesh, out_type=out_t,
                       scratch_types=scratch)
    def gather(uid_h, zid_h, utab_h, ztab_h, uo_h, zo_h,
               ui_v, zi_v, ur_v, zr_v, sem, wsem):
        wid = lax.axis_index("s") * nc + lax.axis_index("c")
        sl = pl.ds(wid * bpw, bpw)
        iu = pltpu.async_copy(uid_h.at[sl], ui_v, sem)
        iz = pltpu.async_copy(zid_h.at[sl], zi_v, sem)
        iu.wait()
        cu = pltpu.async_copy(utab_h.at[ui_v], ur_v, sem)
        iz.wait()
        cz = pltpu.async_copy(ztab_h.at[zi_v], zr_v, sem)
        cu.wait()
        wu = pltpu.async_copy(ur_v, uo_h.at[sl], wsem)
        cz.wait()
        wz = pltpu.async_copy(zr_v, zo_h.at[sl], wsem)
        wu.wait()
        wz.wait()

    return gather(uid, zid, utab, ztab)


def _mlp_body(u, z, g, a, o, gt, at, ot, w1, b1, w2, b2, w3, b3, out):
    gi = g[0, 0, :]
    ai = a[0, 0, :]
    oi = o[0, 0, :]
    goh = (gi[:, None] == lax.broadcasted_iota(jnp.int32, (_BB, 2), 1)
           ).astype(jnp.float32)
    aoh = (ai[:, None] == lax.broadcasted_iota(jnp.int32, (_BB, 7), 1)
           ).astype(jnp.float32)
    ooh = (oi[:, None] == lax.broadcasted_iota(jnp.int32, (_BB, 21), 1)
           ).astype(jnp.float32)
    ge = jnp.dot(goh, gt[...], preferred_element_type=jnp.float32)
    ae = jnp.dot(aoh, at[...], preferred_element_type=jnp.float32)
    oe = jnp.dot(ooh, ot[...], preferred_element_type=jnp.float32)
    x = jnp.concatenate([u[...], ge, ae, oe, z[...]], axis=1)
    h = jnp.dot(x, w1[...], preferred_element_type=jnp.float32) + b1[...]
    h = jnp.maximum(h, 0.0)
    h = jnp.dot(h, w2[...], preferred_element_type=jnp.float32) + b2[...]
    h = jnp.maximum(h, 0.0)
    h = jnp.dot(h, w3[...], preferred_element_type=jnp.float32) + b3[...]
    h = jnp.maximum(h, 0.0)
    ss = jnp.sum(h * h, axis=1, keepdims=True)
    out[...] = h * lax.rsqrt(jnp.maximum(ss, 1e-24))


def _mlp(u, z, gid, aid, oid, gtab, atab, otab, w1, b1, w2, b2, w3, b3):
    nblk = _B // _BB
    const = lambda i: (0, 0)
    return pl.pallas_call(
        _mlp_body,
        grid=(nblk,),
        in_specs=[
            pl.BlockSpec((_BB, _D), lambda i: (i, 0)),
            pl.BlockSpec((_BB, _D), lambda i: (i, 0)),
            pl.BlockSpec((1, 1, _BB), lambda i: (i, 0, 0)),
            pl.BlockSpec((1, 1, _BB), lambda i: (i, 0, 0)),
            pl.BlockSpec((1, 1, _BB), lambda i: (i, 0, 0)),
            pl.BlockSpec((2, _D), const),
            pl.BlockSpec((7, _D), const),
            pl.BlockSpec((21, _D), const),
            pl.BlockSpec((5 * _D, _H1), const),
            pl.BlockSpec((1, _H1), const),
            pl.BlockSpec((_H1, _H2), const),
            pl.BlockSpec((1, _H2), const),
            pl.BlockSpec((_H2, _H3), const),
            pl.BlockSpec((1, _H3), const),
        ],
        out_specs=pl.BlockSpec((_BB, _D), lambda i: (i, 0)),
        out_shape=jax.ShapeDtypeStruct((_B, _D), jnp.float32),
    )(u, z, gid.reshape(nblk, 1, _BB), aid.reshape(nblk, 1, _BB),
      oid.reshape(nblk, 1, _BB), gtab, atab, otab,
      w1, b1.reshape(1, _H1), w2, b2.reshape(1, _H2), w3, b3.reshape(1, _H3))


def kernel(user_id, gender, age, occupation, zip_id, user_tab, gender_tab,
           age_tab, occ_tab, zip_tab, W1, b1, W2, b2, W3, b3):
    uid = user_id.astype(jnp.int32)
    gid = gender.astype(jnp.int32)
    aid = age.astype(jnp.int32)
    oid = occupation.astype(jnp.int32)
    zid = zip_id.astype(jnp.int32)
    u, z = _sc_gather(uid, zid, user_tab, zip_tab)
    return _mlp(u, z, gid, aid, oid, gender_tab, age_tab, occ_tab,
                W1, b1, W2, b2, W3, b3)


# speedup vs baseline: 3.4413x; 1.0131x over previous
import functools

import jax
import jax.numpy as jnp
from jax import lax
from jax.experimental import pallas as pl
from jax.experimental.pallas import tpu as pltpu
from jax.experimental.pallas import tpu_sc as plsc

_B = 4096
_D = 128
_H1 = 1024
_H2 = 512
_H3 = 128
_BB = 512


def _sc_gather(uid, zid, utab, ztab):
    info = plsc.get_sparse_core_info()
    nc, ns = info.num_cores, info.num_subcores
    nw = nc * ns
    bpw = _B // nw

    mesh = plsc.VectorSubcoreMesh(core_axis_name="c", subcore_axis_name="s")
    out_t = [jax.ShapeDtypeStruct((_B, _D), jnp.float32) for _ in range(2)]
    scratch = (
        [pltpu.VMEM((bpw,), jnp.int32) for _ in range(2)]
        + [pltpu.VMEM((bpw, _D), jnp.float32) for _ in range(2)]
        + [pltpu.SemaphoreType.DMA, pltpu.SemaphoreType.DMA]
    )

    @functools.partial(pl.kernel, mesh=mesh, out_type=out_t,
                       scratch_types=scratch)
    def gather(uid_h, zid_h, utab_h, ztab_h, uo_h, zo_h,
               ui_v, zi_v, ur_v, zr_v, sem, wsem):
        wid = lax.axis_index("s") * nc + lax.axis_index("c")
        sl = pl.ds(wid * bpw, bpw)
        iu = pltpu.async_copy(uid_h.at[sl], ui_v, sem)
        iz = pltpu.async_copy(zid_h.at[sl], zi_v, sem)
        iu.wait()
        cu = pltpu.async_copy(utab_h.at[ui_v], ur_v, sem)
        iz.wait()
        cz = pltpu.async_copy(ztab_h.at[zi_v], zr_v, sem)
        cu.wait()
        wu = pltpu.async_copy(ur_v, uo_h.at[sl], wsem)
        cz.wait()
        wz = pltpu.async_copy(zr_v, zo_h.at[sl], wsem)
        wu.wait()
        wz.wait()

    return gather(uid, zid, utab, ztab)


def _mlp_body(u, z, g, a, o, gt, at, ot, w1, b1, w2, b2, w3, b3, out):
    bf = jnp.bfloat16
    gi = g[0, 0, :]
    ai = a[0, 0, :]
    oi = o[0, 0, :]
    goh = (gi[:, None] == lax.broadcasted_iota(jnp.int32, (_BB, 2), 1)
           ).astype(bf)
    aoh = (ai[:, None] == lax.broadcasted_iota(jnp.int32, (_BB, 7), 1)
           ).astype(bf)
    ooh = (oi[:, None] == lax.broadcasted_iota(jnp.int32, (_BB, 21), 1)
           ).astype(bf)
    ge = jnp.dot(goh, gt[...].astype(bf),
                 preferred_element_type=jnp.float32).astype(bf)
    ae = jnp.dot(aoh, at[...].astype(bf),
                 preferred_element_type=jnp.float32).astype(bf)
    oe = jnp.dot(ooh, ot[...].astype(bf),
                 preferred_element_type=jnp.float32).astype(bf)
    x = jnp.concatenate([u[...].astype(bf), ge, ae, oe, z[...].astype(bf)],
                        axis=1)
    h = jnp.dot(x, w1[...], preferred_element_type=jnp.float32) + b1[...]
    h = jnp.maximum(h, 0.0).astype(bf)
    h = jnp.dot(h, w2[...], preferred_element_type=jnp.float32) + b2[...]
    h = jnp.maximum(h, 0.0).astype(bf)
    h = jnp.dot(h, w3[...], preferred_element_type=jnp.float32) + b3[...]
    h = jnp.maximum(h, 0.0)
    ss = jnp.sum(h * h, axis=1, keepdims=True)
    out[...] = h * lax.rsqrt(jnp.maximum(ss, 1e-24))


def _mlp(u, z, gid, aid, oid, gtab, atab, otab, w1, b1, w2, b2, w3, b3):
    nblk = _B // _BB
    const = lambda i: (0, 0)
    return pl.pallas_call(
        _mlp_body,
        grid=(nblk,),
        in_specs=[
            pl.BlockSpec((_BB, _D), lambda i: (i, 0)),
            pl.BlockSpec((_BB, _D), lambda i: (i, 0)),
            pl.BlockSpec((1, 1, _BB), lambda i: (i, 0, 0)),
            pl.BlockSpec((1, 1, _BB), lambda i: (i, 0, 0)),
            pl.BlockSpec((1, 1, _BB), lambda i: (i, 0, 0)),
            pl.BlockSpec((2, _D), const),
            pl.BlockSpec((7, _D), const),
            pl.BlockSpec((21, _D), const),
            pl.BlockSpec((5 * _D, _H1), const),
            pl.BlockSpec((1, _H1), const),
            pl.BlockSpec((_H1, _H2), const),
            pl.BlockSpec((1, _H2), const),
            pl.BlockSpec((_H2, _H3), const),
            pl.BlockSpec((1, _H3), const),
        ],
        out_specs=pl.BlockSpec((_BB, _D), lambda i: (i, 0)),
        out_shape=jax.ShapeDtypeStruct((_B, _D), jnp.float32),
    )(u, z, gid.reshape(nblk, 1, _BB), aid.reshape(nblk, 1, _BB),
      oid.reshape(nblk, 1, _BB), gtab, atab, otab,
      w1.astype(jnp.bfloat16), b1.reshape(1, _H1),
      w2.astype(jnp.bfloat16), b2.reshape(1, _H2),
      w3.astype(jnp.bfloat16), b3.reshape(1, _H3))


def kernel(user_id, gender, age, occupation, zip_id, user_tab, gender_tab,
           age_tab, occ_tab, zip_tab, W1, b1, W2, b2, W3, b3):
    uid = user_id.astype(jnp.int32)
    gid = gender.astype(jnp.int32)
    aid = age.astype(jnp.int32)
    oid = occupation.astype(jnp.int32)
    zid = zip_id.astype(jnp.int32)
    u, z = _sc_gather(uid, zid, user_tab, zip_tab)
    return _mlp(u, z, gid, aid, oid, gender_tab, age_tab, occ_tab,
                W1, b1, W2, b2, W3, b3)


# speedup vs baseline: 3.5276x; 1.0251x over previous
import functools

import jax
import jax.numpy as jnp
from jax import lax
from jax.experimental import pallas as pl
from jax.experimental.pallas import tpu as pltpu
from jax.experimental.pallas import tpu_sc as plsc

_B = 4096
_D = 128
_H1 = 1024
_H2 = 512
_H3 = 128
_BB = 512
_PK = 40


def _sc_gather(uid, zid, utab, ztab):
    info = plsc.get_sparse_core_info()
    nc, ns = info.num_cores, info.num_subcores
    nw = nc * ns
    bpw = _B // nw

    mesh = plsc.VectorSubcoreMesh(core_axis_name="c", subcore_axis_name="s")
    out_t = [jax.ShapeDtypeStruct((_B, _D), jnp.float32) for _ in range(2)]
    scratch = (
        [pltpu.VMEM((bpw,), jnp.int32) for _ in range(2)]
        + [pltpu.VMEM((bpw, _D), jnp.float32) for _ in range(2)]
        + [pltpu.SemaphoreType.DMA, pltpu.SemaphoreType.DMA]
    )

    @functools.partial(pl.kernel, mesh=mesh, out_type=out_t,
                       scratch_types=scratch)
    def gather(uid_h, zid_h, utab_h, ztab_h, uo_h, zo_h,
               ui_v, zi_v, ur_v, zr_v, sem, wsem):
        wid = lax.axis_index("s") * nc + lax.axis_index("c")
        sl = pl.ds(wid * bpw, bpw)
        iu = pltpu.async_copy(uid_h.at[sl], ui_v, sem)
        iz = pltpu.async_copy(zid_h.at[sl], zi_v, sem)
        iu.wait()
        cu = pltpu.async_copy(utab_h.at[ui_v], ur_v, sem)
        iz.wait()
        cz = pltpu.async_copy(ztab_h.at[zi_v], zr_v, sem)
        cu.wait()
        wu = pltpu.async_copy(ur_v, uo_h.at[sl], wsem)
        cz.wait()
        wz = pltpu.async_copy(zr_v, zo_h.at[sl], wsem)
        wu.wait()
        wz.wait()

    return gather(uid, zid, utab, ztab)


def _mlp_body(u, z, g, a, o, gt, at, ot, w1, b1, w2, b2, w3, b3, out,
              w1u_s, w1z_s, p_s, w2_s, w3_s):
    bf = jnp.bfloat16

    @pl.when(pl.program_id(0) == 0)
    def _prep():
        w1u_s[...] = w1[0:_D, :].astype(bf)
        w1z_s[...] = w1[4 * _D:5 * _D, :].astype(bf)
        p_s[...] = jnp.zeros((_PK, _H1), dtype=bf)
        p_s[0:2, :] = jnp.dot(
            gt[...].astype(bf), w1[_D:2 * _D, :].astype(bf),
            preferred_element_type=jnp.float32).astype(bf)
        p_s[8:15, :] = jnp.dot(
            at[...].astype(bf), w1[2 * _D:3 * _D, :].astype(bf),
            preferred_element_type=jnp.float32).astype(bf)
        p_s[16:37, :] = jnp.dot(
            ot[...].astype(bf), w1[3 * _D:4 * _D, :].astype(bf),
            preferred_element_type=jnp.float32).astype(bf)
        w2_s[...] = w2[...].astype(bf)
        w3_s[...] = w3[...].astype(bf)

    gi = g[0, 0, :]
    ai = a[0, 0, :]
    oi = o[0, 0, :]
    i40 = lax.broadcasted_iota(jnp.int32, (_BB, _PK), 1)
    coh = ((gi[:, None] == i40) | (ai[:, None] + 8 == i40)
           | (oi[:, None] + 16 == i40)).astype(bf)
    h = (jnp.dot(u[...].astype(bf), w1u_s[...],
                 preferred_element_type=jnp.float32)
         + jnp.dot(z[...].astype(bf), w1z_s[...],
                   preferred_element_type=jnp.float32)
         + jnp.dot(coh, p_s[...], preferred_element_type=jnp.float32)
         + b1[...])
    h = jnp.maximum(h, 0.0).astype(bf)
    h = jnp.dot(h, w2_s[...], preferred_element_type=jnp.float32) + b2[...]
    h = jnp.maximum(h, 0.0).astype(bf)
    h = jnp.dot(h, w3_s[...], preferred_element_type=jnp.float32) + b3[...]
    h = jnp.maximum(h, 0.0)
    ss = jnp.sum(h * h, axis=1, keepdims=True)
    out[...] = h * lax.rsqrt(jnp.maximum(ss, 1e-24))


def _mlp(u, z, gid, aid, oid, gtab, atab, otab, w1, b1, w2, b2, w3, b3):
    nblk = _B // _BB
    const = lambda i: (0, 0)
    return pl.pallas_call(
        _mlp_body,
        grid=(nblk,),
        in_specs=[
            pl.BlockSpec((_BB, _D), lambda i: (i, 0)),
            pl.BlockSpec((_BB, _D), lambda i: (i, 0)),
            pl.BlockSpec((1, 1, _BB), lambda i: (i, 0, 0)),
            pl.BlockSpec((1, 1, _BB), lambda i: (i, 0, 0)),
            pl.BlockSpec((1, 1, _BB), lambda i: (i, 0, 0)),
            pl.BlockSpec((2, _D), const),
            pl.BlockSpec((7, _D), const),
            pl.BlockSpec((21, _D), const),
            pl.BlockSpec((5 * _D, _H1), const),
            pl.BlockSpec((1, _H1), const),
            pl.BlockSpec((_H1, _H2), const),
            pl.BlockSpec((1, _H2), const),
            pl.BlockSpec((_H2, _H3), const),
            pl.BlockSpec((1, _H3), const),
        ],
        out_specs=pl.BlockSpec((_BB, _D), lambda i: (i, 0)),
        out_shape=jax.ShapeDtypeStruct((_B, _D), jnp.float32),
        scratch_shapes=[
            pltpu.VMEM((_D, _H1), jnp.bfloat16),
            pltpu.VMEM((_D, _H1), jnp.bfloat16),
            pltpu.VMEM((_PK, _H1), jnp.bfloat16),
            pltpu.VMEM((_H1, _H2), jnp.bfloat16),
            pltpu.VMEM((_H2, _H3), jnp.bfloat16),
        ],
    )(u, z, gid.reshape(nblk, 1, _BB), aid.reshape(nblk, 1, _BB),
      oid.reshape(nblk, 1, _BB), gtab, atab, otab,
      w1, b1.reshape(1, _H1), w2, b2.reshape(1, _H2), w3, b3.reshape(1, _H3))


def kernel(user_id, gender, age, occupation, zip_id, user_tab, gender_tab,
           age_tab, occ_tab, zip_tab, W1, b1, W2, b2, W3, b3):
    uid = user_id.astype(jnp.int32)
    gid = gender.astype(jnp.int32)
    aid = age.astype(jnp.int32)
    oid = occupation.astype(jnp.int32)
    zid = zip_id.astype(jnp.int32)
    u, z = _sc_gather(uid, zid, user_tab, zip_tab)
    return _mlp(u, z, gid, aid, oid, gender_tab, age_tab, occ_tab,
                W1, b1, W2, b2, W3, b3)


# speedup vs baseline: 3.6144x; 1.0246x over previous
import functools

import jax
import jax.numpy as jnp
from jax import lax
from jax.experimental import pallas as pl
from jax.experimental.pallas import tpu as pltpu
from jax.experimental.pallas import tpu_sc as plsc

_B = 4096
_D = 128
_H1 = 1024
_H2 = 512
_H3 = 128
_BB = 1024
_PK = 40


def _sc_gather(uid, zid, utab, ztab):
    info = plsc.get_sparse_core_info()
    nc, ns = info.num_cores, info.num_subcores
    nw = nc * ns
    bpw = _B // nw

    mesh = plsc.VectorSubcoreMesh(core_axis_name="c", subcore_axis_name="s")
    out_t = [jax.ShapeDtypeStruct((_B, _D), jnp.float32) for _ in range(2)]
    scratch = (
        [pltpu.VMEM((bpw,), jnp.int32) for _ in range(2)]
        + [pltpu.VMEM((bpw, _D), jnp.float32) for _ in range(2)]
        + [pltpu.SemaphoreType.DMA, pltpu.SemaphoreType.DMA]
    )

    @functools.partial(pl.kernel, mesh=mesh, out_type=out_t,
                       scratch_types=scratch)
    def gather(uid_h, zid_h, utab_h, ztab_h, uo_h, zo_h,
               ui_v, zi_v, ur_v, zr_v, sem, wsem):
        wid = lax.axis_index("s") * nc + lax.axis_index("c")
        sl = pl.ds(wid * bpw, bpw)
        iu = pltpu.async_copy(uid_h.at[sl], ui_v, sem)
        iz = pltpu.async_copy(zid_h.at[sl], zi_v, sem)
        iu.wait()
        cu = pltpu.async_copy(utab_h.at[ui_v], ur_v, sem)
        iz.wait()
        cz = pltpu.async_copy(ztab_h.at[zi_v], zr_v, sem)
        cu.wait()
        wu = pltpu.async_copy(ur_v, uo_h.at[sl], wsem)
        cz.wait()
        wz = pltpu.async_copy(zr_v, zo_h.at[sl], wsem)
        wu.wait()
        wz.wait()

    return gather(uid, zid, utab, ztab)


def _mlp_body(u, z, g, a, o, gt, at, ot, w1u, w1m, w1z, b1, w2, b2, w3, b3,
              out, p_s):
    bf = jnp.bfloat16

    @pl.when(pl.program_id(0) == 0)
    def _prep():
        p_s[...] = jnp.zeros((_PK, _H1), dtype=bf)
        p_s[0:2, :] = jnp.dot(
            gt[...].astype(bf), w1m[0:_D, :],
            preferred_element_type=jnp.float32).astype(bf)
        p_s[8:15, :] = jnp.dot(
            at[...].astype(bf), w1m[_D:2 * _D, :],
            preferred_element_type=jnp.float32).astype(bf)
        p_s[16:37, :] = jnp.dot(
            ot[...].astype(bf), w1m[2 * _D:3 * _D, :],
            preferred_element_type=jnp.float32).astype(bf)

    gi = g[0, 0, :]
    ai = a[0, 0, :]
    oi = o[0, 0, :]
    i40 = lax.broadcasted_iota(jnp.int32, (_BB, _PK), 1)
    coh = ((gi[:, None] == i40) | (ai[:, None] + 8 == i40)
           | (oi[:, None] + 16 == i40)).astype(bf)
    h = (jnp.dot(u[...].astype(bf), w1u[...],
                 preferred_element_type=jnp.float32)
         + jnp.dot(z[...].astype(bf), w1z[...],
                   preferred_element_type=jnp.float32)
         + jnp.dot(coh, p_s[...], preferred_element_type=jnp.float32)
         + b1[...])
    h = jnp.maximum(h, 0.0).astype(bf)
    h = jnp.dot(h, w2[...], preferred_element_type=jnp.float32) + b2[...]
    h = jnp.maximum(h, 0.0).astype(bf)
    h = jnp.dot(h, w3[...], preferred_element_type=jnp.float32) + b3[...]
    h = jnp.maximum(h, 0.0)
    ss = jnp.sum(h * h, axis=1, keepdims=True)
    out[...] = h * lax.rsqrt(jnp.maximum(ss, 1e-24))


def _mlp(u, z, gid, aid, oid, gtab, atab, otab, w1, b1, w2, b2, w3, b3):
    nblk = _B // _BB
    bf = jnp.bfloat16
    const = lambda i: (0, 0)
    w1b = w1.astype(bf)
    return pl.pallas_call(
        _mlp_body,
        grid=(nblk,),
        in_specs=[
            pl.BlockSpec((_BB, _D), lambda i: (i, 0)),
            pl.BlockSpec((_BB, _D), lambda i: (i, 0)),
            pl.BlockSpec((1, 1, _BB), lambda i: (i, 0, 0)),
            pl.BlockSpec((1, 1, _BB), lambda i: (i, 0, 0)),
            pl.BlockSpec((1, 1, _BB), lambda i: (i, 0, 0)),
            pl.BlockSpec((2, _D), const),
            pl.BlockSpec((7, _D), const),
            pl.BlockSpec((21, _D), const),
            pl.BlockSpec((_D, _H1), const),
            pl.BlockSpec((3 * _D, _H1), const),
            pl.BlockSpec((_D, _H1), const),
            pl.BlockSpec((1, _H1), const),
            pl.BlockSpec((_H1, _H2), const),
            pl.BlockSpec((1, _H2), const),
            pl.BlockSpec((_H2, _H3), const),
            pl.BlockSpec((1, _H3), const),
        ],
        out_specs=pl.BlockSpec((_BB, _D), lambda i: (i, 0)),
        out_shape=jax.ShapeDtypeStruct((_B, _D), jnp.float32),
        scratch_shapes=[
            pltpu.VMEM((_PK, _H1), jnp.bfloat16),
        ],
    )(u, z, gid.reshape(nblk, 1, _BB), aid.reshape(nblk, 1, _BB),
      oid.reshape(nblk, 1, _BB), gtab, atab, otab,
      w1b[0:_D], w1b[_D:4 * _D], w1b[4 * _D:5 * _D], b1.reshape(1, _H1),
      w2.astype(bf), b2.reshape(1, _H2), w3.astype(bf), b3.reshape(1, _H3))


def kernel(user_id, gender, age, occupation, zip_id, user_tab, gender_tab,
           age_tab, occ_tab, zip_tab, W1, b1, W2, b2, W3, b3):
    uid = user_id.astype(jnp.int32)
    gid = gender.astype(jnp.int32)
    aid = age.astype(jnp.int32)
    oid = occupation.astype(jnp.int32)
    zid = zip_id.astype(jnp.int32)
    u, z = _sc_gather(uid, zid, user_tab, zip_tab)
    return _mlp(u, z, gid, aid, oid, gender_tab, age_tab, occ_tab,
                W1, b1, W2, b2, W3, b3)


# speedup vs baseline: 3.6391x; 1.0069x over previous
import functools

import jax
import jax.numpy as jnp
from jax import lax
from jax.experimental import pallas as pl
from jax.experimental.pallas import tpu as pltpu
from jax.experimental.pallas import tpu_sc as plsc

_B = 4096
_D = 128
_H1 = 1024
_H2 = 512
_H3 = 128
_BB = 1024
_PK = 40


def _sc_gather(uid, zid, utab, ztab):
    info = plsc.get_sparse_core_info()
    nc, ns = info.num_cores, info.num_subcores
    nw = nc * ns
    bpw = _B // nw

    mesh = plsc.VectorSubcoreMesh(core_axis_name="c", subcore_axis_name="s")
    out_t = jax.ShapeDtypeStruct((_B, 2 * _D), jnp.float32)
    scratch = (
        [pltpu.VMEM((bpw,), jnp.int32) for _ in range(2)]
        + [pltpu.VMEM((bpw, _D), jnp.float32) for _ in range(2)]
        + [pltpu.SemaphoreType.DMA, pltpu.SemaphoreType.DMA]
    )

    @functools.partial(pl.kernel, mesh=mesh, out_type=out_t,
                       scratch_types=scratch)
    def gather(idx_h, utab_h, ztab_h, o_h,
               ui_v, zi_v, ur_v, zr_v, sem, wsem):
        wid = lax.axis_index("s") * nc + lax.axis_index("c")
        base = wid * bpw
        sl = pl.ds(base, bpw)
        iu = pltpu.async_copy(idx_h.at[0, sl], ui_v, sem)
        iz = pltpu.async_copy(idx_h.at[1, sl], zi_v, sem)
        iu.wait()
        cu = pltpu.async_copy(utab_h.at[ui_v], ur_v, sem)
        iz.wait()
        cz = pltpu.async_copy(ztab_h.at[zi_v], zr_v, sem)
        cu.wait()
        wu = pltpu.async_copy(ur_v, o_h.at[sl, pl.ds(0, _D)], wsem)
        cz.wait()
        wz = pltpu.async_copy(zr_v, o_h.at[sl, pl.ds(_D, _D)], wsem)
        wu.wait()
        wz.wait()

    idx = jnp.stack([uid, zid])
    return gather(idx, utab, ztab)


def _mlp_body(uz, g, a, o, gt, at, ot, w1u, w1m, w1z, b1, w2, b2, w3, b3,
              out, p_s):
    bf = jnp.bfloat16

    @pl.when(pl.program_id(0) == 0)
    def _prep():
        p_s[...] = jnp.zeros((_PK, _H1), dtype=bf)
        p_s[0:2, :] = jnp.dot(
            gt[...].astype(bf), w1m[0:_D, :],
            preferred_element_type=jnp.float32).astype(bf)
        p_s[8:15, :] = jnp.dot(
            at[...].astype(bf), w1m[_D:2 * _D, :],
            preferred_element_type=jnp.float32).astype(bf)
        p_s[16:37, :] = jnp.dot(
            ot[...].astype(bf), w1m[2 * _D:3 * _D, :],
            preferred_element_type=jnp.float32).astype(bf)

    gi = g[0, 0, :]
    ai = a[0, 0, :]
    oi = o[0, 0, :]
    i40 = lax.broadcasted_iota(jnp.int32, (_BB, _PK), 1)
    coh = ((gi[:, None] == i40) | (ai[:, None] + 8 == i40)
           | (oi[:, None] + 16 == i40)).astype(bf)
    uzb = uz[...].astype(bf)
    h = (jnp.dot(uzb[:, :_D], w1u[...],
                 preferred_element_type=jnp.float32)
         + jnp.dot(uzb[:, _D:], w1z[...],
                   preferred_element_type=jnp.float32)
         + jnp.dot(coh, p_s[...], preferred_element_type=jnp.float32)
         + b1[...])
    h = jnp.maximum(h, 0.0).astype(bf)
    h = jnp.dot(h, w2[...], preferred_element_type=jnp.float32) + b2[...]
    h = jnp.maximum(h, 0.0).astype(bf)
    h = jnp.dot(h, w3[...], preferred_element_type=jnp.float32) + b3[...]
    h = jnp.maximum(h, 0.0)
    ss = jnp.sum(h * h, axis=1, keepdims=True)
    out[...] = h * lax.rsqrt(jnp.maximum(ss, 1e-24))


def _mlp(uz, gid, aid, oid, gtab, atab, otab, w1, b1, w2, b2, w3, b3):
    nblk = _B // _BB
    bf = jnp.bfloat16
    const = lambda i: (0, 0)
    w1b = w1.astype(bf)
    return pl.pallas_call(
        _mlp_body,
        grid=(nblk,),
        in_specs=[
            pl.BlockSpec((_BB, 2 * _D), lambda i: (i, 0)),
            pl.BlockSpec((1, 1, _BB), lambda i: (i, 0, 0)),
            pl.BlockSpec((1, 1, _BB), lambda i: (i, 0, 0)),
            pl.BlockSpec((1, 1, _BB), lambda i: (i, 0, 0)),
            pl.BlockSpec((2, _D), const),
            pl.BlockSpec((7, _D), const),
            pl.BlockSpec((21, _D), const),
            pl.BlockSpec((_D, _H1), const),
            pl.BlockSpec((3 * _D, _H1), const),
            pl.BlockSpec((_D, _H1), const),
            pl.BlockSpec((1, _H1), const),
            pl.BlockSpec((_H1, _H2), const),
            pl.BlockSpec((1, _H2), const),
            pl.BlockSpec((_H2, _H3), const),
            pl.BlockSpec((1, _H3), const),
        ],
        out_specs=pl.BlockSpec((_BB, _D), lambda i: (i, 0)),
        out_shape=jax.ShapeDtypeStruct((_B, _D), jnp.float32),
        scratch_shapes=[
            pltpu.VMEM((_PK, _H1), jnp.bfloat16),
        ],
    )(uz, gid.reshape(nblk, 1, _BB), aid.reshape(nblk, 1, _BB),
      oid.reshape(nblk, 1, _BB), gtab, atab, otab,
      w1b[0:_D], w1b[_D:4 * _D], w1b[4 * _D:5 * _D], b1.reshape(1, _H1),
      w2.astype(bf), b2.reshape(1, _H2), w3.astype(bf), b3.reshape(1, _H3))


def kernel(user_id, gender, age, occupation, zip_id, user_tab, gender_tab,
           age_tab, occ_tab, zip_tab, W1, b1, W2, b2, W3, b3):
    uid = user_id.astype(jnp.int32)
    gid = gender.astype(jnp.int32)
    aid = age.astype(jnp.int32)
    oid = occupation.astype(jnp.int32)
    zid = zip_id.astype(jnp.int32)
    uz = _sc_gather(uid, zid, user_tab, zip_tab)
    return _mlp(uz, gid, aid, oid, gender_tab, age_tab, occ_tab,
                W1, b1, W2, b2, W3, b3)


# speedup vs baseline: 3.6454x; 1.0017x over previous
import functools

import jax
import jax.numpy as jnp
from jax import lax
from jax.experimental import pallas as pl
from jax.experimental.pallas import tpu as pltpu
from jax.experimental.pallas import tpu_sc as plsc

_B = 4096
_D = 128
_H1 = 1024
_H2 = 512
_H3 = 128
_BB = 1024
_PK = 40


def _sc_gather(uid, zid, utab, ztab):
    info = plsc.get_sparse_core_info()
    nc, ns = info.num_cores, info.num_subcores
    nw = nc * ns
    bpw = _B // nw

    mesh = plsc.VectorSubcoreMesh(core_axis_name="c", subcore_axis_name="s")
    out_t = jax.ShapeDtypeStruct((_B, 2 * _D), jnp.float32)
    scratch = (
        [pltpu.VMEM((bpw,), jnp.int32) for _ in range(2)]
        + [pltpu.VMEM((bpw, _D), jnp.float32) for _ in range(2)]
        + [pltpu.SemaphoreType.DMA, pltpu.SemaphoreType.DMA]
    )

    nch = 4
    ch = bpw // nch

    @functools.partial(pl.kernel, mesh=mesh, out_type=out_t,
                       scratch_types=scratch)
    def gather(uid_h, zid_h, utab_h, ztab_h, o_h,
               ui_v, zi_v, ur_v, zr_v, sem, wsem):
        wid = lax.axis_index("s") * nc + lax.axis_index("c")
        base = wid * bpw
        sl = pl.ds(base, bpw)
        iu = pltpu.async_copy(uid_h.at[sl], ui_v, sem)
        iz = pltpu.async_copy(zid_h.at[sl], zi_v, sem)
        iu.wait()
        iz.wait()
        gs = []
        for c in range(nch):
            cs = pl.ds(c * ch, ch)
            gs.append(pltpu.async_copy(utab_h.at[ui_v.at[cs]], ur_v.at[cs],
                                       sem))
            gs.append(pltpu.async_copy(ztab_h.at[zi_v.at[cs]], zr_v.at[cs],
                                       sem))
        ws = []
        for c in range(nch):
            osl = pl.ds(base + c * ch, ch)
            cs = pl.ds(c * ch, ch)
            gs[2 * c].wait()
            ws.append(pltpu.async_copy(ur_v.at[cs],
                                       o_h.at[osl, pl.ds(0, _D)], wsem))
            gs[2 * c + 1].wait()
            ws.append(pltpu.async_copy(zr_v.at[cs],
                                       o_h.at[osl, pl.ds(_D, _D)], wsem))
        for w in ws:
            w.wait()

    return gather(uid, zid, utab, ztab)


def _mlp_body(uz, g, a, o, gt, at, ot, w1u, w1m, w1z, b1, w2, b2, w3, b3,
              out, p_s):
    bf = jnp.bfloat16

    @pl.when(pl.program_id(0) == 0)
    def _prep():
        p_s[...] = jnp.zeros((_PK, _H1), dtype=bf)
        p_s[0:2, :] = jnp.dot(
            gt[...].astype(bf), w1m[0:_D, :],
            preferred_element_type=jnp.float32).astype(bf)
        p_s[8:15, :] = jnp.dot(
            at[...].astype(bf), w1m[_D:2 * _D, :],
            preferred_element_type=jnp.float32).astype(bf)
        p_s[16:37, :] = jnp.dot(
            ot[...].astype(bf), w1m[2 * _D:3 * _D, :],
            preferred_element_type=jnp.float32).astype(bf)

    gi = g[0, 0, :]
    ai = a[0, 0, :]
    oi = o[0, 0, :]
    i40 = lax.broadcasted_iota(jnp.int32, (_BB, _PK), 1)
    coh = ((gi[:, None] == i40) | (ai[:, None] + 8 == i40)
           | (oi[:, None] + 16 == i40)).astype(bf)
    uzb = uz[...].astype(bf)
    h = (jnp.dot(uzb[:, :_D], w1u[...],
                 preferred_element_type=jnp.float32)
         + jnp.dot(uzb[:, _D:], w1z[...],
                   preferred_element_type=jnp.float32)
         + jnp.dot(coh, p_s[...], preferred_element_type=jnp.float32)
         + b1[...])
    h = jnp.maximum(h, 0.0).astype(bf)
    h = jnp.dot(h, w2[...], preferred_element_type=jnp.float32) + b2[...]
    h = jnp.maximum(h, 0.0).astype(bf)
    h = jnp.dot(h, w3[...], preferred_element_type=jnp.float32) + b3[...]
    h = jnp.maximum(h, 0.0)
    ss = jnp.sum(h * h, axis=1, keepdims=True)
    out[...] = h * lax.rsqrt(jnp.maximum(ss, 1e-24))


def _mlp(uz, gid, aid, oid, gtab, atab, otab, w1, b1, w2, b2, w3, b3):
    nblk = _B // _BB
    bf = jnp.bfloat16
    const = lambda i: (0, 0)
    w1b = w1.astype(bf)
    return pl.pallas_call(
        _mlp_body,
        grid=(nblk,),
        in_specs=[
            pl.BlockSpec((_BB, 2 * _D), lambda i: (i, 0)),
            pl.BlockSpec((1, 1, _BB), lambda i: (i, 0, 0)),
            pl.BlockSpec((1, 1, _BB), lambda i: (i, 0, 0)),
            pl.BlockSpec((1, 1, _BB), lambda i: (i, 0, 0)),
            pl.BlockSpec((2, _D), const),
            pl.BlockSpec((7, _D), const),
            pl.BlockSpec((21, _D), const),
            pl.BlockSpec((_D, _H1), const),
            pl.BlockSpec((3 * _D, _H1), const),
            pl.BlockSpec((_D, _H1), const),
            pl.BlockSpec((1, _H1), const),
            pl.BlockSpec((_H1, _H2), const),
            pl.BlockSpec((1, _H2), const),
            pl.BlockSpec((_H2, _H3), const),
            pl.BlockSpec((1, _H3), const),
        ],
        out_specs=pl.BlockSpec((_BB, _D), lambda i: (i, 0)),
        out_shape=jax.ShapeDtypeStruct((_B, _D), jnp.float32),
        scratch_shapes=[
            pltpu.VMEM((_PK, _H1), jnp.bfloat16),
        ],
    )(uz, gid.reshape(nblk, 1, _BB), aid.reshape(nblk, 1, _BB),
      oid.reshape(nblk, 1, _BB), gtab, atab, otab,
      w1b[0:_D], w1b[_D:4 * _D], w1b[4 * _D:5 * _D], b1.reshape(1, _H1),
      w2.astype(bf), b2.reshape(1, _H2), w3.astype(bf), b3.reshape(1, _H3))


def kernel(user_id, gender, age, occupation, zip_id, user_tab, gender_tab,
           age_tab, occ_tab, zip_tab, W1, b1, W2, b2, W3, b3):
    uid = user_id.astype(jnp.int32)
    gid = gender.astype(jnp.int32)
    aid = age.astype(jnp.int32)
    oid = occupation.astype(jnp.int32)
    zid = zip_id.astype(jnp.int32)
    uz = _sc_gather(uid, zid, user_tab, zip_tab)
    return _mlp(uz, gid, aid, oid, gender_tab, age_tab, occ_tab,
                W1, b1, W2, b2, W3, b3)
